# baseline (speedup 1.0000x reference)
import functools

import jax
import jax.numpy as jnp
from jax import lax
from jax.experimental import pallas as pl
from jax.experimental.pallas import tpu as pltpu
from jax.experimental.pallas import tpu_sc as plsc

B, N = 4, 4096
D = 768
H = 768
M = B * N
POS = 10240
NC, NS = 2, 16
NW = NC * NS
MPW = M // NW
C = 32
NCHUNK = MPW // C
IPW = MPW * 2


def _pe_gather(table2, idx2):
    mesh = plsc.VectorSubcoreMesh(core_axis_name="c", subcore_axis_name="s")

    @functools.partial(
        pl.kernel,
        mesh=mesh,
        out_type=jax.ShapeDtypeStruct((M, H), jnp.float32),
        scratch_types=[
            pltpu.VMEM((IPW,), jnp.int32),
            [pltpu.VMEM((2 * C, H), jnp.float32) for _ in range(2)],
            [pltpu.SemaphoreType.DMA for _ in range(2)],
            [pltpu.SemaphoreType.DMA for _ in range(2)],
        ],
    )
    def k(tab_hbm, idx_hbm, out_hbm, idxv, rows, gsem, ssem):
        wid = lax.axis_index("s") * NC + lax.axis_index("c")
        base = wid * MPW
        pltpu.sync_copy(idx_hbm.at[pl.ds(wid * IPW, IPW)], idxv)

        def gather_start(j, b):
            pltpu.async_copy(
                tab_hbm.at[idxv.at[pl.ds(j * 2 * C, 2 * C)]], rows[b], gsem[b]
            )

        def gather_wait(b):
            pltpu.make_async_copy(
                tab_hbm.at[idxv.at[pl.ds(0, 2 * C)]], rows[b], gsem[b]
            ).wait()

        def scatter_start(j, b):
            pltpu.async_copy(
                rows[b].at[pl.ds(0, C)], out_hbm.at[pl.ds(base + j * C, C)],
                ssem[b],
            )

        def scatter_wait(b):
            pltpu.make_async_copy(
                rows[b].at[pl.ds(0, C)], out_hbm.at[pl.ds(base, C)], ssem[b]
            ).wait()

        def add_rows(b):
            def add_row(r, c2):
                for c in range(H // 16):
                    sl = pl.ds(c * 16, 16)
                    rows[b][r, sl] = rows[b][r, sl] + rows[b][C + r, sl]
                return c2

            lax.fori_loop(0, C, add_row, 0)

        gather_start(0, 0)

        def pair(j2, carry):
            for b in range(2):
                j = j2 * 2 + b
                nb = 1 - b
                if b == 0:
                    @pl.when(j2 > 0)
                    def _():
                        scatter_wait(nb)

                    gather_start(j + 1, nb)
                else:
                    scatter_wait(nb)

                    @pl.when(j2 < NCHUNK // 2 - 1)
                    def _():
                        gather_start(j + 1, nb)

                gather_wait(b)
                add_rows(b)
                scatter_start(j, b)
            return carry

        lax.fori_loop(0, NCHUNK // 2, pair, 0)
        scatter_wait(1)

    return k(table2, idx2)


BM = 1024


def _mm_body(px_ref, w_ref, pe_ref, out_ref):
    pxn = 2.0 * px_ref[...] - 1.0
    acc = lax.dot_general(
        pxn,
        w_ref[...],
        (((1,), (1,)), ((), ())),
        preferred_element_type=jnp.float32,
        precision=lax.Precision.DEFAULT,
    )
    out_ref[...] = acc + pe_ref[...]


def _mm(px, w, pe):
    return pl.pallas_call(
        _mm_body,
        grid=(M // BM,),
        in_specs=[
            pl.BlockSpec((BM, D), lambda i: (i, 0)),
            pl.BlockSpec((H, D), lambda i: (0, 0)),
            pl.BlockSpec((BM, H), lambda i: (i, 0)),
        ],
        out_specs=pl.BlockSpec((BM, H), lambda i: (i, 0)),
        out_shape=jax.ShapeDtypeStruct((M, H), jnp.float32),
    )(px, w, pe)


def kernel(pixel_values, pixel_position_ids, padding_mask, W, pos_table):
    del padding_mask
    px = pixel_values.reshape(M, D)
    table2 = pos_table.reshape(2 * POS, H)
    ids = pixel_position_ids.reshape(M, 2)
    ix = ids[:, 0].reshape(M // C, C)
    iy = ids[:, 1].reshape(M // C, C) + POS
    idx2 = jnp.stack([ix, iy], axis=1).reshape(2 * M)
    pe = _pe_gather(table2, idx2)
    h = _mm(px, W, pe)
    return h.reshape(B, N, H)


# speedup vs baseline: 1.0373x; 1.0373x over previous
import functools

import jax
import jax.numpy as jnp
from jax import lax
from jax.experimental import pallas as pl
from jax.experimental.pallas import tpu as pltpu
from jax.experimental.pallas import tpu_sc as plsc

B, N = 4, 4096
D = 768
H = 768
M = B * N
POS = 10240
NC, NS = 2, 16
NW = NC * NS
S = 4
MG = M // S
MPW = MG // NW
C = 64
NCHUNK = MPW // C
IPW = MPW * 2


def _pe_gather(table2, idx2g):
    mesh = plsc.VectorSubcoreMesh(core_axis_name="c", subcore_axis_name="s")

    @functools.partial(
        pl.kernel,
        mesh=mesh,
        out_type=jax.ShapeDtypeStruct((MG, H), jnp.float32),
        scratch_types=[
            pltpu.VMEM((IPW,), jnp.int32),
            pltpu.VMEM((2 * C, H), jnp.float32),
            pltpu.SemaphoreType.DMA,
        ],
    )
    def k(tab_hbm, idx_hbm, out_hbm, idxv, rows, sem):
        wid = lax.axis_index("s") * NC + lax.axis_index("c")
        pltpu.sync_copy(idx_hbm.at[pl.ds(wid * IPW, IPW)], idxv)

        def chunk(j, carry):
            pltpu.async_copy(
                tab_hbm.at[idxv.at[pl.ds(j * 2 * C, 2 * C)]], rows, sem
            ).wait()

            def add_row(r, c2):
                for c in range(H // 16):
                    sl = pl.ds(c * 16, 16)
                    rows[r, sl] = rows[r, sl] + rows[C + r, sl]
                return c2

            lax.fori_loop(0, C, add_row, 0)
            off = wid * MPW + j * C
            pltpu.sync_copy(rows.at[pl.ds(0, C)], out_hbm.at[pl.ds(off, C)])
            return carry

        lax.fori_loop(0, NCHUNK, chunk, 0)

    return k(table2, idx2g)


BM = 1024
GB = MG // BM


def _mm_body(px_ref, w_ref, pe_ref, out_ref):
    pxn = 2.0 * px_ref[...] - 1.0
    acc = lax.dot_general(
        pxn,
        w_ref[...],
        (((1,), (1,)), ((), ())),
        preferred_element_type=jnp.float32,
        precision=lax.Precision.DEFAULT,
    )
    out_ref[...] = acc + pe_ref[...]


def _mm_body_acc(px_ref, w_ref, pe_ref, h_ref, out_ref):
    del h_ref
    _mm_body(px_ref, w_ref, pe_ref, out_ref)


def _mm_group(g, px, w, pe_g, h):
    out_spec = pl.BlockSpec((BM, H), lambda i, g=g: (g * GB + i, 0))
    in_specs = [
        pl.BlockSpec((BM, D), lambda i, g=g: (g * GB + i, 0)),
        pl.BlockSpec((H, D), lambda i: (0, 0)),
        pl.BlockSpec((BM, H), lambda i: (i, 0)),
    ]
    if g == 0:
        return pl.pallas_call(
            _mm_body,
            grid=(GB,),
            in_specs=in_specs,
            out_specs=out_spec,
            out_shape=jax.ShapeDtypeStruct((M, H), jnp.float32),
        )(px, w, pe_g)
    return pl.pallas_call(
        _mm_body_acc,
        grid=(GB,),
        in_specs=in_specs + [pl.BlockSpec(memory_space=pl.ANY)],
        out_specs=out_spec,
        out_shape=jax.ShapeDtypeStruct((M, H), jnp.float32),
        input_output_aliases={3: 0},
    )(px, w, pe_g, h)


def kernel(pixel_values, pixel_position_ids, padding_mask, W, pos_table):
    del padding_mask
    px = pixel_values.reshape(M, D)
    table2 = pos_table.reshape(2 * POS, H)
    ids = pixel_position_ids.reshape(M, 2)
    ix = ids[:, 0].reshape(M // C, C)
    iy = ids[:, 1].reshape(M // C, C) + POS
    idx2 = jnp.stack([ix, iy], axis=1).reshape(2 * M)
    pes = [
        _pe_gather(table2, lax.slice(idx2, (g * 2 * MG,), ((g + 1) * 2 * MG,)))
        for g in range(S)
    ]
    h = None
    for g in range(S):
        h = _mm_group(g, px, W, pes[g], h)
    return h.reshape(B, N, H)
